# Initial kernel scaffold; baseline (speedup 1.0000x reference)
import jax
import jax.numpy as jnp
from jax.experimental import pallas as pl


def kernel(skills, language_operators, W0, b0, W1, b1, W2, b2, codebook):
    raise NotImplementedError("write your pallas kernel here")


# speedup vs baseline: 2.1366x; 2.1366x over previous
import functools

import jax
import jax.numpy as jnp
from jax import lax
from jax.experimental import pallas as pl
from jax.experimental.pallas import tpu as pltpu

_B = 1024
_K = 1024
_D = 64
_BB = 128
_KC = 256


def _vq_body(x_ref, w0_ref, b0_ref, w1_ref, b1_ref, w2_ref, b2_ref, cb_ref,
             u_ref, q_ref):
    x = x_ref[...]
    h = jnp.maximum(
        jnp.dot(x, w0_ref[...], preferred_element_type=jnp.float32) + b0_ref[...], 0.0)
    h = jnp.maximum(
        jnp.dot(h, w1_ref[...], preferred_element_type=jnp.float32) + b1_ref[...], 0.0)
    u = jnp.dot(h, w2_ref[...], preferred_element_type=jnp.float32) + b2_ref[...]
    u_ref[...] = u

    u_aug = jnp.concatenate((u * -2.0, jnp.ones((_BB, 1), jnp.float32)), axis=1)

    def dist_step(i, carry):
        best_d, best_i = carry
        cb_c = cb_ref[pl.ds(i * _KC, _KC), :]
        c2 = jnp.sum(cb_c * cb_c, axis=1, keepdims=True)
        cb_aug = jnp.concatenate((cb_c, c2), axis=1)
        d = lax.dot_general(u_aug, cb_aug, (((1,), (1,)), ((), ())),
                            preferred_element_type=jnp.float32,
                            precision=lax.Precision.HIGHEST)
        dmin = jnp.min(d, axis=1, keepdims=True)
        iota = lax.broadcasted_iota(jnp.int32, d.shape, 1) + i * _KC
        imin = jnp.min(jnp.where(d == dmin, iota, _K), axis=1, keepdims=True)
        take = dmin < best_d
        return (jnp.where(take, dmin, best_d), jnp.where(take, imin, best_i))

    init = (jnp.full((_BB, 1), jnp.inf, jnp.float32),
            jnp.zeros((_BB, 1), jnp.int32))
    _, idx = lax.fori_loop(0, _K // _KC, dist_step, init)

    def gather_step(i, q):
        cb_c = cb_ref[pl.ds(i * _KC, _KC), :]
        iota = lax.broadcasted_iota(jnp.int32, (_BB, _KC), 1) + i * _KC
        onehot = (iota == idx).astype(jnp.float32)
        return q + jnp.dot(onehot, cb_c, preferred_element_type=jnp.float32,
                           precision=lax.Precision.HIGHEST)

    q_ref[...] = lax.fori_loop(0, _K // _KC, gather_step,
                               jnp.zeros((_BB, _D), jnp.float32))


def kernel(skills, language_operators, W0, b0, W1, b1, W2, b2, codebook):
    x = jnp.concatenate((skills, language_operators), axis=-1)
    nsteps = _B // _BB
    row_block = lambda i: (i, 0)
    whole = lambda i: (0, 0)
    u, q = pl.pallas_call(
        _vq_body,
        grid=(nsteps,),
        in_specs=[
            pl.BlockSpec((_BB, 640), row_block),
            pl.BlockSpec((640, 256), whole),
            pl.BlockSpec((1, 256), whole),
            pl.BlockSpec((256, 256), whole),
            pl.BlockSpec((1, 256), whole),
            pl.BlockSpec((256, _D), whole),
            pl.BlockSpec((1, _D), whole),
            pl.BlockSpec((_K, _D), whole),
        ],
        out_specs=(
            pl.BlockSpec((_BB, _D), row_block),
            pl.BlockSpec((_BB, _D), row_block),
        ),
        out_shape=(
            jax.ShapeDtypeStruct((_B, _D), jnp.float32),
            jax.ShapeDtypeStruct((_B, _D), jnp.float32),
        ),
        compiler_params=pltpu.CompilerParams(
            dimension_semantics=("arbitrary",),
        ),
    )(x, W0, b0[None, :], W1, b1[None, :], W2, b2[None, :], codebook)
    return (u, q)
